# baseline (speedup 1.0000x reference)
import jax
import jax.numpy as jnp
from jax import lax
from jax.experimental import pallas as pl
from jax.experimental.pallas import tpu as pltpu

_V = 100000
_D = 640
_NCTX = 20
_ED = 32
_TILE = 4000
_NT = _V // _TILE


def _gather_body(idx_smem, emb_hbm, o_hbm, sem):
    copies = []
    for k in range(_NCTX):
        base = (idx_smem[k] // 4) * 128
        cp = pltpu.make_async_copy(
            emb_hbm.at[pl.ds(base, 128)],
            o_hbm.at[pl.ds(k * 128, 128)],
            sem,
        )
        cp.start()
        copies.append(cp)
    for cp in copies:
        cp.wait()


def _gather(inputs, emb):
    grid_spec = pltpu.PrefetchScalarGridSpec(
        num_scalar_prefetch=1,
        grid=(1,),
        in_specs=[pl.BlockSpec(memory_space=pl.ANY)],
        out_specs=pl.BlockSpec(memory_space=pl.ANY),
        scratch_shapes=[pltpu.SemaphoreType.DMA],
    )
    return pl.pallas_call(
        _gather_body,
        grid_spec=grid_spec,
        out_shape=jax.ShapeDtypeStruct((_NCTX * 128,), jnp.float32),
    )(inputs, emb.reshape(_V * _ED))


_H = _TILE // 2


def _mv_body(idx_smem, er_ref, wa_ref, wb_ref, b_ref, out_ref, e_ref,
             acc_ref):
    i = pl.program_id(0)

    @pl.when(i == 0)
    def _():
        acc_ref[0] = -jnp.inf
        acc_ref[1] = 0.0
        pieces = []
        for k in range(_NCTX):
            p = idx_smem[k] % 4
            r = er_ref[k:k + 1, :]
            piece = jnp.where(
                p == 0, r[:, 0:32],
                jnp.where(p == 1, r[:, 32:64],
                          jnp.where(p == 2, r[:, 64:96], r[:, 96:128])))
            pieces.append(piece)
        e_ref[...] = jnp.concatenate(pieces, axis=1)

    dn = (((1,), (1,)), ((), ()))
    ta = lax.dot_general(e_ref[...], wa_ref[...], dn,
                         preferred_element_type=jnp.float32)
    tb = lax.dot_general(e_ref[...], wb_ref[...], dn,
                         preferred_element_type=jnp.float32)
    ta = ta + b_ref[0, :, 0:_H]
    tb = tb + b_ref[0, :, _H:_TILE]
    out_ref[pl.ds(i, 1), 0:_H] = ta
    out_ref[pl.ds(i, 1), _H:_TILE] = tb

    m_prev = acc_ref[0]
    s_prev = acc_ref[1]
    m_new = jnp.maximum(m_prev,
                        jnp.maximum(jnp.max(ta), jnp.max(tb)))
    s_new = (s_prev * jnp.exp(m_prev - m_new)
             + jnp.sum(jnp.exp(ta - m_new))
             + jnp.sum(jnp.exp(tb - m_new)))
    acc_ref[0] = m_new
    acc_ref[1] = s_new

    @pl.when(i == _NT - 1)
    def _():
        out_ref[...] = out_ref[...] - (m_new + jnp.log(s_new))


def _matvec_logsoftmax(inputs, er, W1, b1r):
    grid_spec = pltpu.PrefetchScalarGridSpec(
        num_scalar_prefetch=1,
        grid=(_NT,),
        in_specs=[
            pl.BlockSpec((_NCTX, 128), lambda i, idx: (0, 0)),
            pl.BlockSpec((_H, _D), lambda i, idx: (2 * i, 0)),
            pl.BlockSpec((_H, _D), lambda i, idx: (2 * i + 1, 0)),
            pl.BlockSpec((1, 1, _TILE), lambda i, idx: (i, 0, 0)),
        ],
        out_specs=pl.BlockSpec((_NT, _TILE), lambda i, idx: (0, 0)),
        scratch_shapes=[pltpu.VMEM((1, _D), jnp.float32),
                        pltpu.SMEM((2,), jnp.float32)],
    )
    return pl.pallas_call(
        _mv_body,
        grid_spec=grid_spec,
        out_shape=jax.ShapeDtypeStruct((_NT, _TILE), jnp.float32),
    )(inputs, er, W1, W1, b1r)


def kernel(inputs, emb, W1, b1):
    er = _gather(inputs, emb).reshape(_NCTX, 128)
    b1r = b1.reshape(_NT, 1, _TILE)
    log_probs = _matvec_logsoftmax(inputs, er, W1, b1r)
    return log_probs.reshape(1, _V)


# speedup vs baseline: 1.2158x; 1.2158x over previous
import jax
import jax.numpy as jnp
from jax import lax
from jax.experimental import pallas as pl
from jax.experimental.pallas import tpu as pltpu

_V = 100000
_D = 640
_NCTX = 20
_ED = 32
_TILE = 4000
_NT = _V // _TILE


def _gather_body(idx_smem, emb_blk, o_ref):
    k = pl.program_id(0)
    r = idx_smem[k] % 8
    o_ref[...] = emb_blk[pl.ds(r, 1), :].reshape(1, 1, _ED)


def _gather(inputs, emb):
    grid_spec = pltpu.PrefetchScalarGridSpec(
        num_scalar_prefetch=1,
        grid=(_NCTX,),
        in_specs=[pl.BlockSpec((8, _ED), lambda k, idx: (idx[k] // 8, 0))],
        out_specs=pl.BlockSpec((1, 1, _ED), lambda k, idx: (k, 0, 0)),
    )
    return pl.pallas_call(
        _gather_body,
        grid_spec=grid_spec,
        out_shape=jax.ShapeDtypeStruct((_NCTX, 1, _ED), jnp.float32),
    )(inputs, emb)


_H = _TILE // 2


def _mv_body(er_ref, wa_ref, wb_ref, b_ref, out_ref, e_ref, acc_ref):
    i = pl.program_id(0)

    @pl.when(i == 0)
    def _():
        acc_ref[0] = -jnp.inf
        acc_ref[1] = 0.0
        e_ref[...] = jnp.concatenate(
            [er_ref[k:k + 1, :] for k in range(_NCTX)], axis=1)

    dn = (((1,), (1,)), ((), ()))
    ta = lax.dot_general(e_ref[...], wa_ref[...], dn,
                         preferred_element_type=jnp.float32)
    tb = lax.dot_general(e_ref[...], wb_ref[...], dn,
                         preferred_element_type=jnp.float32)
    ta = ta + b_ref[0, :, 0:_H]
    tb = tb + b_ref[0, :, _H:_TILE]
    out_ref[pl.ds(i, 1), 0:_H] = ta
    out_ref[pl.ds(i, 1), _H:_TILE] = tb

    m_prev = acc_ref[0]
    s_prev = acc_ref[1]
    m_new = jnp.maximum(m_prev,
                        jnp.maximum(jnp.max(ta), jnp.max(tb)))
    s_new = (s_prev * jnp.exp(m_prev - m_new)
             + jnp.sum(jnp.exp(ta - m_new))
             + jnp.sum(jnp.exp(tb - m_new)))
    acc_ref[0] = m_new
    acc_ref[1] = s_new

    @pl.when(i == _NT - 1)
    def _():
        out_ref[...] = out_ref[...] - (m_new + jnp.log(s_new))


def _matvec_logsoftmax(e, W1, b1r):
    return pl.pallas_call(
        _mv_body,
        grid=(_NT,),
        in_specs=[
            pl.BlockSpec((_NCTX, _ED), lambda i: (0, 0)),
            pl.BlockSpec((_H, _D), lambda i: (2 * i, 0)),
            pl.BlockSpec((_H, _D), lambda i: (2 * i + 1, 0)),
            pl.BlockSpec((1, 1, _TILE), lambda i: (i, 0, 0)),
        ],
        out_specs=pl.BlockSpec((_NT, _TILE), lambda i: (0, 0)),
        out_shape=jax.ShapeDtypeStruct((_NT, _TILE), jnp.float32),
        scratch_shapes=[pltpu.VMEM((1, _D), jnp.float32),
                        pltpu.SMEM((2,), jnp.float32)],
    )(e, W1, W1, b1r)


def kernel(inputs, emb, W1, b1):
    e = _gather(inputs, emb).reshape(_NCTX, _ED)
    b1r = b1.reshape(_NT, 1, _TILE)
    log_probs = _matvec_logsoftmax(e, W1, b1r)
    return log_probs.reshape(1, _V)


# speedup vs baseline: 1.7494x; 1.4389x over previous
import jax
import jax.numpy as jnp
from jax import lax
from jax.experimental import pallas as pl
from jax.experimental.pallas import tpu as pltpu

_V = 100000
_D = 640
_NCTX = 20
_ED = 32
_TILE = 4000
_NT = _V // _TILE


_H = _TILE // 2


def _mv_body(idx_smem, embT_hbm, wa_ref, wb_ref, b_ref, out_ref, e_ref,
             blk_ref, acc_ref, sem):
    i = pl.program_id(0)

    @pl.when(i == 0)
    def _():
        acc_ref[0] = -jnp.inf
        acc_ref[1] = 0.0
        copies = []
        for k in range(_NCTX):
            col = idx_smem[k]
            base = (col // 128) * 128
            cp = pltpu.make_async_copy(
                embT_hbm.at[:, pl.ds(base, 128)],
                blk_ref.at[k],
                sem,
            )
            cp.start()
            copies.append(cp)
        for cp in copies:
            cp.wait()
        lanes = lax.broadcasted_iota(jnp.int32, (1, 128), 1)
        dn = (((1,), (1,)), ((), ()))
        pieces = []
        for k in range(_NCTX):
            oh = (lanes == idx_smem[k] % 128).astype(jnp.float32)
            pieces.append(lax.dot_general(
                oh, blk_ref[k], dn, preferred_element_type=jnp.float32))
        e_ref[...] = jnp.concatenate(pieces, axis=1)

    dn = (((1,), (1,)), ((), ()))
    ta = lax.dot_general(e_ref[...], wa_ref[...], dn,
                         preferred_element_type=jnp.float32)
    tb = lax.dot_general(e_ref[...], wb_ref[...], dn,
                         preferred_element_type=jnp.float32)
    ta = ta + b_ref[0, :, 0:_H]
    tb = tb + b_ref[0, :, _H:_TILE]
    out_ref[pl.ds(i, 1), 0:_H] = ta
    out_ref[pl.ds(i, 1), _H:_TILE] = tb

    m_prev = acc_ref[0]
    s_prev = acc_ref[1]
    m_new = jnp.maximum(m_prev,
                        jnp.maximum(jnp.max(ta), jnp.max(tb)))
    s_new = (s_prev * jnp.exp(m_prev - m_new)
             + jnp.sum(jnp.exp(ta - m_new))
             + jnp.sum(jnp.exp(tb - m_new)))
    acc_ref[0] = m_new
    acc_ref[1] = s_new

    @pl.when(i == _NT - 1)
    def _():
        out_ref[...] = out_ref[...] - (m_new + jnp.log(s_new))


def _matvec_logsoftmax(inputs, embT, W1, b1r):
    grid_spec = pltpu.PrefetchScalarGridSpec(
        num_scalar_prefetch=1,
        grid=(_NT,),
        in_specs=[
            pl.BlockSpec(memory_space=pl.ANY),
            pl.BlockSpec((_H, _D), lambda i, idx: (2 * i, 0)),
            pl.BlockSpec((_H, _D), lambda i, idx: (2 * i + 1, 0)),
            pl.BlockSpec((1, 1, _TILE), lambda i, idx: (i, 0, 0)),
        ],
        out_specs=pl.BlockSpec((_NT, _TILE), lambda i, idx: (0, 0)),
        scratch_shapes=[
            pltpu.VMEM((1, _D), jnp.float32),
            pltpu.VMEM((_NCTX, _ED, 128), jnp.float32),
            pltpu.SMEM((2,), jnp.float32),
            pltpu.SemaphoreType.DMA,
        ],
    )
    return pl.pallas_call(
        _mv_body,
        grid_spec=grid_spec,
        out_shape=jax.ShapeDtypeStruct((_NT, _TILE), jnp.float32),
    )(inputs, embT, W1, W1, b1r)


def kernel(inputs, emb, W1, b1):
    b1r = b1.reshape(_NT, 1, _TILE)
    log_probs = _matvec_logsoftmax(inputs, emb.T, W1, b1r)
    return log_probs.reshape(1, _V)
